# baseline (speedup 1.0000x reference)
import functools

import jax
import jax.numpy as jnp
from jax import lax
from jax.experimental import pallas as pl
from jax.experimental.pallas import tpu as pltpu
from jax.experimental.pallas import tpu_sc as plsc

EMBED_DIM = 32
BATCH = 16384
LANES = 16

NUM_CORES = 2
NUM_SUBCORES = 16
NUM_WORKERS = NUM_CORES * NUM_SUBCORES
B_PER_W = BATCH // NUM_WORKERS
CHUNK = 128
N_CHUNK = B_PER_W // CHUNK


def _dot_body(uids_hbm, iids_hbm, utab_hbm, itab_hbm, out_hbm,
              uid_v, iid_v, urows, irows, out_v, sem):
    wid = lax.axis_index("s") * NUM_CORES + lax.axis_index("c")
    base = wid * B_PER_W
    idx_row = wid * N_CHUNK

    pltpu.sync_copy(uids_hbm.at[pl.ds(idx_row, N_CHUNK)], uid_v)
    pltpu.sync_copy(iids_hbm.at[pl.ds(idx_row, N_CHUNK)], iid_v)

    copies = []
    for j in range(N_CHUNK):
        copies.append(pltpu.async_copy(
            utab_hbm.at[uid_v.at[j]], urows.at[pl.ds(j * CHUNK, CHUNK)], sem))
        copies.append(pltpu.async_copy(
            itab_hbm.at[iid_v.at[j]], irows.at[pl.ds(j * CHUNK, CHUNK)], sem))
    for c in copies:
        c.wait()

    lane = lax.iota(jnp.int32, LANES)

    def body(g, _):
        rows = g * LANES + lane
        acc = jnp.zeros((LANES,), jnp.float32)
        for d in range(EMBED_DIM):
            col = jnp.full((LANES,), d, jnp.int32)
            uc = plsc.load_gather(urows, [rows, col])
            vc = plsc.load_gather(irows, [rows, col])
            acc = acc + uc * vc
        out_v[pl.ds(g * LANES, LANES)] = acc
        return 0

    lax.fori_loop(0, B_PER_W // LANES, body, 0)

    pltpu.sync_copy(out_v, out_hbm.at[pl.ds(base, B_PER_W)])


_sc_call = functools.partial(
    pl.kernel,
    mesh=plsc.VectorSubcoreMesh(core_axis_name="c", subcore_axis_name="s"),
    out_type=jax.ShapeDtypeStruct((BATCH,), jnp.float32),
    compiler_params=pltpu.CompilerParams(
        needs_layout_passes=False, use_tc_tiling_on_sc=False),
    scratch_types=[
        pltpu.VMEM((N_CHUNK, CHUNK), jnp.int32),
        pltpu.VMEM((N_CHUNK, CHUNK), jnp.int32),
        pltpu.VMEM((B_PER_W, EMBED_DIM), jnp.float32),
        pltpu.VMEM((B_PER_W, EMBED_DIM), jnp.float32),
        pltpu.VMEM((B_PER_W,), jnp.float32),
        pltpu.SemaphoreType.DMA,
    ],
)(_dot_body)


@jax.jit
def kernel(user_ids, item_ids, user_table, item_table):
    uids = jnp.asarray(user_ids, jnp.int32).reshape(NUM_WORKERS * N_CHUNK, CHUNK)
    iids = jnp.asarray(item_ids, jnp.int32).reshape(NUM_WORKERS * N_CHUNK, CHUNK)
    return _sc_call(uids, iids, user_table, item_table)
